# Initial kernel scaffold; baseline (speedup 1.0000x reference)
import jax
import jax.numpy as jnp
from jax.experimental import pallas as pl


def kernel(z_e, embedding):
    raise NotImplementedError("write your pallas kernel here")


# speedup vs baseline: 1.0318x; 1.0318x over previous
import jax
import jax.numpy as jnp
from jax.experimental import pallas as pl
from jax.experimental.pallas import tpu as pltpu

_K = 1024
_C = 256
_B = 16
_T = 1024
_BETA = 0.25


def _vq_body(z_ref, emb_ref, loss_ref, zq_ref, perp_ref, enc_ref,
             sumsq_ref, counts_ref):
    b = pl.program_id(0)
    nb = pl.num_programs(0)

    zb = z_ref[0]
    emb = emb_ref[...]

    inner = jax.lax.dot_general(
        emb.astype(jnp.bfloat16), zb.astype(jnp.bfloat16),
        (((1,), (0,)), ((), ())),
        preferred_element_type=jnp.float32)
    e_l2 = jnp.sum(emb * emb, axis=1, keepdims=True)
    z_l2 = jnp.sum(zb * zb, axis=0, keepdims=True)
    dist = z_l2 + e_l2 - 2.0 * inner

    iota_kt = jax.lax.broadcasted_iota(jnp.int32, (_K, _T), 0)
    mn = jnp.min(dist, axis=0, keepdims=True)
    idx = jnp.min(jnp.where(dist == mn, iota_kt, _K), axis=0)

    onehot_kt = (iota_kt == idx[None, :]).astype(jnp.float32)

    zq = jax.lax.dot_general(
        emb.astype(jnp.bfloat16), onehot_kt.astype(jnp.bfloat16),
        (((0,), (0,)), ((), ())),
        preferred_element_type=jnp.float32)
    zq_ref[0] = zq

    iota_tk = jax.lax.broadcasted_iota(jnp.int32, (_T, _K), 1)
    enc_ref[...] = (iota_tk == idx[:, None]).astype(jnp.float32)

    diff = zq - zb
    part = jnp.sum(diff * diff)
    cnt = jnp.sum(onehot_kt, axis=1)[None, :]

    @pl.when(b == 0)
    def _init():
        sumsq_ref[0, 0] = part
        counts_ref[...] = cnt

    @pl.when(b > 0)
    def _acc():
        sumsq_ref[0, 0] += part
        counts_ref[...] += cnt

    @pl.when(b == nb - 1)
    def _final():
        n_elem = _B * _T * _C
        loss_ref[0] = (1.0 + _BETA) * sumsq_ref[0, 0] / float(n_elem)
        p = counts_ref[...] / float(_B * _T)
        perp_ref[0] = jnp.exp(-jnp.sum(p * jnp.log(p + 1e-10)))


def kernel(z_e, embedding):
    z3 = z_e.reshape(_B, _C, _T)

    out_types = (
        jax.ShapeDtypeStruct((1,), jnp.float32),
        jax.ShapeDtypeStruct((_B, _C, _T), jnp.float32),
        jax.ShapeDtypeStruct((1,), jnp.float32),
        jax.ShapeDtypeStruct((_B * _T, _K), jnp.float32),
    )

    loss, zq3, perp, enc = pl.pallas_call(
        _vq_body,
        grid=(_B,),
        in_specs=[
            pl.BlockSpec((1, _C, _T), lambda b: (b, 0, 0)),
            pl.BlockSpec((_K, _C), lambda b: (0, 0)),
        ],
        out_specs=(
            pl.BlockSpec(memory_space=pltpu.SMEM),
            pl.BlockSpec((1, _C, _T), lambda b: (b, 0, 0)),
            pl.BlockSpec(memory_space=pltpu.SMEM),
            pl.BlockSpec((_T, _K), lambda b: (b, 0)),
        ),
        out_shape=out_types,
        scratch_shapes=[
            pltpu.SMEM((1, 1), jnp.float32),
            pltpu.VMEM((1, _K), jnp.float32),
        ],
    )(z3, embedding)

    z_q_st = zq3.reshape(_B, _C, 32, 32)
    return (loss[0], z_q_st, perp[0], enc)


# speedup vs baseline: 1.1131x; 1.0788x over previous
import jax
import jax.numpy as jnp
from jax.experimental import pallas as pl
from jax.experimental.pallas import tpu as pltpu

_K = 1024
_C = 256
_B = 16
_T = 1024
_BETA = 0.25


def _vq_body(z_ref, emb_ref, loss_ref, zq_ref, perp_ref, enc_ref,
             sumsq_ref, counts_ref):
    b = pl.program_id(0)
    nb = pl.num_programs(0)

    zb = z_ref[0]
    emb = emb_ref[...]

    inner = jax.lax.dot_general(
        emb.astype(jnp.bfloat16), zb.astype(jnp.bfloat16),
        (((1,), (0,)), ((), ())),
        preferred_element_type=jnp.float32)
    e_l2 = jnp.sum(emb * emb, axis=1, keepdims=True)
    z_l2 = jnp.sum(zb * zb, axis=0, keepdims=True)
    dist = z_l2 + e_l2 - 2.0 * inner

    iota_kt = jax.lax.broadcasted_iota(jnp.int32, (_K, _T), 0)
    mn = jnp.min(dist, axis=0, keepdims=True)
    idx = jnp.min(jnp.where(dist == mn, iota_kt, _K), axis=0)

    onehot_kt = (iota_kt == idx[None, :]).astype(jnp.float32)

    zq = jax.lax.dot_general(
        emb.astype(jnp.bfloat16), onehot_kt.astype(jnp.bfloat16),
        (((0,), (0,)), ((), ())),
        preferred_element_type=jnp.float32)
    zq_ref[0] = zq

    enc_ref[...] = jnp.swapaxes(onehot_kt, 0, 1)

    diff = zq - zb
    part = jnp.sum(diff * diff)
    cnt = jnp.sum(onehot_kt, axis=1, keepdims=True)

    @pl.when(b == 0)
    def _init():
        sumsq_ref[0, 0] = part
        counts_ref[...] = cnt

    @pl.when(b > 0)
    def _acc():
        sumsq_ref[0, 0] += part
        counts_ref[...] += cnt

    @pl.when(b == nb - 1)
    def _final():
        n_elem = _B * _T * _C
        loss_ref[0] = (1.0 + _BETA) * sumsq_ref[0, 0] / float(n_elem)
        p = counts_ref[...] / float(_B * _T)
        perp_ref[0] = jnp.exp(-jnp.sum(p * jnp.log(p + 1e-10)))


def kernel(z_e, embedding):
    z3 = z_e.reshape(_B, _C, _T)

    out_types = (
        jax.ShapeDtypeStruct((1,), jnp.float32),
        jax.ShapeDtypeStruct((_B, _C, _T), jnp.float32),
        jax.ShapeDtypeStruct((1,), jnp.float32),
        jax.ShapeDtypeStruct((_B * _T, _K), jnp.float32),
    )

    loss, zq3, perp, enc = pl.pallas_call(
        _vq_body,
        grid=(_B,),
        in_specs=[
            pl.BlockSpec((1, _C, _T), lambda b: (b, 0, 0)),
            pl.BlockSpec((_K, _C), lambda b: (0, 0)),
        ],
        out_specs=(
            pl.BlockSpec(memory_space=pltpu.SMEM),
            pl.BlockSpec((1, _C, _T), lambda b: (b, 0, 0)),
            pl.BlockSpec(memory_space=pltpu.SMEM),
            pl.BlockSpec((_T, _K), lambda b: (b, 0)),
        ),
        out_shape=out_types,
        scratch_shapes=[
            pltpu.SMEM((1, 1), jnp.float32),
            pltpu.VMEM((_K, 1), jnp.float32),
        ],
    )(z3, embedding)

    z_q_st = zq3.reshape(_B, _C, 32, 32)
    return (loss[0], z_q_st, perp[0], enc)
